# baseline (speedup 1.0000x reference)
import jax
import jax.numpy as jnp
from jax.experimental import pallas as pl
from jax.experimental.pallas import tpu as pltpu

BM = 256
CK = 1024


def _body(t_ref, x_ref, o_ref, xt_ref, cbuf_ref, tbuf_ref, obuf_ref,
          xsems, tsems, osems):
    B = x_ref.shape[0]
    C = x_ref.shape[2]
    V_out, V_in = t_ref.shape
    nchunk = V_in // CK
    nm = V_out // BM

    def x_copy(i, slot):
        return pltpu.make_async_copy(
            x_ref.at[:, pl.ds(i * CK, CK), :],
            cbuf_ref.at[slot],
            xsems.at[slot],
        )

    def t_copy(m, slot):
        return pltpu.make_async_copy(
            t_ref.at[pl.ds(m * BM, BM), :],
            tbuf_ref.at[slot],
            tsems.at[slot],
        )

    def o_copy(m, slot):
        return pltpu.make_async_copy(
            obuf_ref.at[slot],
            o_ref.at[:, pl.ds(m * BM, BM), :],
            osems.at[slot],
        )

    def stage_out(m, partial):
        slot = m % 2
        if m >= 2:
            o_copy(m - 2, slot).wait()
        for b in range(B):
            obuf_ref[slot, b, :, :] = partial[:, b * C:(b + 1) * C]
        o_copy(m, slot).start()

    t_copy(0, 0).start()
    t_copy(1, 1).start()
    t_copy(2, 2).start()
    x_copy(0, 0).start()

    t_copy(0, 0).wait()
    acc = None
    for i in range(nchunk):
        slot = i % 2
        if i + 1 < nchunk:
            x_copy(i + 1, (i + 1) % 2).start()
        x_copy(i, slot).wait()
        for b in range(B):
            xt_ref[pl.ds(i * CK, CK), b * C:(b + 1) * C] = (
                cbuf_ref[slot, b].astype(jnp.bfloat16))
        tc = tbuf_ref[0][:, i * CK:(i + 1) * CK].astype(jnp.bfloat16)
        d = jnp.dot(tc, xt_ref[pl.ds(i * CK, CK), :],
                    preferred_element_type=jnp.float32)
        acc = d if acc is None else acc + d
    stage_out(0, acc)

    for m in range(1, nm):
        slot = m % 3
        t_copy(m, slot).wait()
        if m + 3 < nm + 1:
            t_copy(m + 2, (m + 2) % 3).start()
        t = tbuf_ref[slot].astype(jnp.bfloat16)
        stage_out(m, jnp.dot(t, xt_ref[...],
                             preferred_element_type=jnp.float32))

    o_copy(nm - 2, (nm - 2) % 2).wait()
    o_copy(nm - 1, (nm - 1) % 2).wait()


@jax.jit
def kernel(x, transform):
    B, V_in, C = x.shape
    V_out = transform.shape[0]
    N = B * C

    return pl.pallas_call(
        _body,
        grid=(1,),
        in_specs=[
            pl.BlockSpec(memory_space=pltpu.MemorySpace.HBM),
            pl.BlockSpec(memory_space=pltpu.MemorySpace.HBM),
        ],
        out_specs=pl.BlockSpec(memory_space=pltpu.MemorySpace.HBM),
        out_shape=jax.ShapeDtypeStruct((B, V_out, C), jnp.float32),
        scratch_shapes=[
            pltpu.VMEM((V_in, N), jnp.bfloat16),
            pltpu.VMEM((2, B, CK, C), jnp.float32),
            pltpu.VMEM((3, BM, V_in), jnp.float32),
            pltpu.VMEM((2, B, BM, C), jnp.float32),
            pltpu.SemaphoreType.DMA((2,)),
            pltpu.SemaphoreType.DMA((3,)),
            pltpu.SemaphoreType.DMA((2,)),
        ],
        compiler_params=pltpu.CompilerParams(
            dimension_semantics=("arbitrary",),
        ),
    )(transform, x)


# speedup vs baseline: 1.0344x; 1.0344x over previous
import jax
import jax.numpy as jnp
from jax.experimental import pallas as pl
from jax.experimental.pallas import tpu as pltpu

BM = 256
CK = 1024


def _body(t_ref, x_ref, o_ref, xt_ref, cbuf_ref, tbuf_ref, obuf_ref,
          xsems, tsems, osems):
    B = x_ref.shape[0]
    C = x_ref.shape[2]
    V_out, V_in = t_ref.shape
    nchunk = V_in // CK
    nm = V_out // BM

    def x_copy(i, slot):
        return pltpu.make_async_copy(
            x_ref.at[:, pl.ds(i * CK, CK), :],
            cbuf_ref.at[slot],
            xsems.at[slot],
        )

    def t_copy(m, slot):
        return pltpu.make_async_copy(
            t_ref.at[pl.ds(m * BM, BM), :],
            tbuf_ref.at[slot],
            tsems.at[slot],
        )

    def o_copy(m, slot):
        return pltpu.make_async_copy(
            obuf_ref.at[slot],
            o_ref.at[:, pl.ds(m * BM, BM), :],
            osems.at[slot],
        )

    def stage_out(m, partial):
        slot = m % 2
        if m >= 2:
            o_copy(m - 2, slot).wait()
        for b in range(B):
            obuf_ref[slot, b, :, :] = partial[:, b * C:(b + 1) * C]
        o_copy(m, slot).start()

    t_copy(0, 0).start()
    t_copy(1, 1).start()
    x_copy(0, 0).start()
    t_copy(0, 0).wait()
    t_copy(1, 1).wait()

    acc0 = None
    acc1 = None
    for i in range(nchunk):
        slot = i % 2
        if i + 1 < nchunk:
            x_copy(i + 1, (i + 1) % 2).start()
        x_copy(i, slot).wait()
        for b in range(B):
            xt_ref[pl.ds(i * CK, CK), b * C:(b + 1) * C] = (
                cbuf_ref[slot, b].astype(jnp.bfloat16))
        xc = xt_ref[pl.ds(i * CK, CK), :]
        d0 = jnp.dot(tbuf_ref[0][:, i * CK:(i + 1) * CK].astype(jnp.bfloat16),
                     xc, preferred_element_type=jnp.float32)
        acc0 = d0 if acc0 is None else acc0 + d0
        d1 = jnp.dot(tbuf_ref[1][:, i * CK:(i + 1) * CK].astype(jnp.bfloat16),
                     xc, preferred_element_type=jnp.float32)
        acc1 = d1 if acc1 is None else acc1 + d1
    t_copy(2, 2).start()
    stage_out(0, acc0)
    stage_out(1, acc1)

    for m in range(2, nm):
        slot = m % 3
        t_copy(m, slot).wait()
        if m + 1 < nm:
            t_copy(m + 1, (m + 1) % 3).start()
        t = tbuf_ref[slot].astype(jnp.bfloat16)
        stage_out(m, jnp.dot(t, xt_ref[...],
                             preferred_element_type=jnp.float32))

    o_copy(nm - 2, (nm - 2) % 2).wait()
    o_copy(nm - 1, (nm - 1) % 2).wait()


@jax.jit
def kernel(x, transform):
    B, V_in, C = x.shape
    V_out = transform.shape[0]
    N = B * C

    return pl.pallas_call(
        _body,
        grid=(1,),
        in_specs=[
            pl.BlockSpec(memory_space=pltpu.MemorySpace.HBM),
            pl.BlockSpec(memory_space=pltpu.MemorySpace.HBM),
        ],
        out_specs=pl.BlockSpec(memory_space=pltpu.MemorySpace.HBM),
        out_shape=jax.ShapeDtypeStruct((B, V_out, C), jnp.float32),
        scratch_shapes=[
            pltpu.VMEM((V_in, N), jnp.bfloat16),
            pltpu.VMEM((2, B, CK, C), jnp.float32),
            pltpu.VMEM((3, BM, V_in), jnp.float32),
            pltpu.VMEM((2, B, BM, C), jnp.float32),
            pltpu.SemaphoreType.DMA((2,)),
            pltpu.SemaphoreType.DMA((3,)),
            pltpu.SemaphoreType.DMA((2,)),
        ],
        compiler_params=pltpu.CompilerParams(
            dimension_semantics=("arbitrary",),
        ),
    )(transform, x)
